# baseline (speedup 1.0000x reference)
import functools

import jax
import jax.numpy as jnp
from jax.experimental import pallas as pl
from jax.experimental.pallas import tpu as pltpu

N = 196
D = 768
B = 64
TH = 0.8
K2 = 3
MIN_N = 20
SCALE = 5.0
BIG = 2**30


def _entropies(aff):
    e = jnp.exp(aff * SCALE)
    ae = aff * e
    sq = jnp.sum(e, axis=1)
    wq = jnp.sum(ae, axis=1)
    sr = jnp.sum(e, axis=0)
    wr = jnp.sum(ae, axis=0)
    ent_q = jnp.log(sq) - SCALE * wq / sq
    ent_r = jnp.log(sr) - SCALE * wr / sr
    return ent_q, ent_r


def _adaptive_mask(ent):
    emin = jnp.min(ent)
    emax = jnp.max(ent)
    abs_th = emin + (1.0 - TH) * (emax - emin)
    mask = ent < abs_th
    count = jnp.sum(mask.astype(jnp.int32))
    e_i = ent[:, None]
    e_j = ent[None, :]
    ii = jax.lax.broadcasted_iota(jnp.int32, (N, N), 0)
    jj = jax.lax.broadcasted_iota(jnp.int32, (N, N), 1)
    smaller = (e_j < e_i) | ((e_j == e_i) & (jj < ii))
    ranks = jnp.sum(smaller.astype(jnp.int32), axis=1)
    fallback = (ranks < MIN_N).astype(jnp.int32)
    return jnp.where(count < MIN_N, fallback, mask.astype(jnp.int32))


def _top3(aff, axis):
    idx_iota = jax.lax.broadcasted_iota(jnp.int32, (N, N), axis)
    work = aff
    cols = []
    for _ in range(K2):
        m = jnp.max(work, axis=axis, keepdims=True)
        idx = jnp.min(jnp.where(work == m, idx_iota, BIG), axis=axis)
        cols.append(idx[:, None])
        sel = idx[:, None] if axis == 1 else idx[None, :]
        work = jnp.where(idx_iota == sel, -jnp.inf, work)
    return jnp.concatenate(cols, axis=1)


def _image_kernel(q_ref, r_ref, aff_ref, entq_ref, entr_ref,
                  maskq_ref, maskr_ref, knnq_ref, knnr_ref):
    q = q_ref[0]
    r = r_ref[0]
    qn = q / jnp.maximum(jnp.sqrt(jnp.sum(q * q, axis=1, keepdims=True)), 1e-12)
    rn = r / jnp.maximum(jnp.sqrt(jnp.sum(r * r, axis=1, keepdims=True)), 1e-12)
    aff = jax.lax.dot_general(qn, rn, (((1,), (1,)), ((), ())),
                              preferred_element_type=jnp.float32)
    aff_ref[0] = aff

    ent_q, ent_r = _entropies(aff)
    entq_ref[0, 0] = ent_q
    entr_ref[0, 0] = ent_r

    maskq_ref[0, 0] = _adaptive_mask(ent_q)
    maskr_ref[0, 0] = _adaptive_mask(ent_r)

    knnq_ref[0] = _top3(aff, axis=1)
    knnr_ref[0] = _top3(aff, axis=0)


@jax.jit
def kernel(que_tokens, ref_tokens):
    grid = (B,)
    in_specs = [
        pl.BlockSpec((1, N, D), lambda b: (b, 0, 0)),
        pl.BlockSpec((1, N, D), lambda b: (b, 0, 0)),
    ]
    out_specs = [
        pl.BlockSpec((1, N, N), lambda b: (b, 0, 0)),
        pl.BlockSpec((1, 1, N), lambda b: (b, 0, 0)),
        pl.BlockSpec((1, 1, N), lambda b: (b, 0, 0)),
        pl.BlockSpec((1, 1, N), lambda b: (b, 0, 0)),
        pl.BlockSpec((1, 1, N), lambda b: (b, 0, 0)),
        pl.BlockSpec((1, N, K2), lambda b: (b, 0, 0)),
        pl.BlockSpec((1, N, K2), lambda b: (b, 0, 0)),
    ]
    out_shapes = [
        jax.ShapeDtypeStruct((B, N, N), jnp.float32),
        jax.ShapeDtypeStruct((B, 1, N), jnp.float32),
        jax.ShapeDtypeStruct((B, 1, N), jnp.float32),
        jax.ShapeDtypeStruct((B, 1, N), jnp.int32),
        jax.ShapeDtypeStruct((B, 1, N), jnp.int32),
        jax.ShapeDtypeStruct((B, N, K2), jnp.int32),
        jax.ShapeDtypeStruct((B, N, K2), jnp.int32),
    ]
    aff, ent_q, ent_r, mask_q, mask_r, knn_q2r, knn_r2q = pl.pallas_call(
        _image_kernel,
        grid=grid,
        in_specs=in_specs,
        out_specs=out_specs,
        out_shape=out_shapes,
        compiler_params=pltpu.CompilerParams(
            dimension_semantics=("parallel",),
        ),
    )(que_tokens, ref_tokens)
    return (aff,
            ent_q.reshape(B, N),
            ent_r.reshape(B, N),
            mask_q.reshape(B, N).astype(jnp.bool_),
            mask_r.reshape(B, N).astype(jnp.bool_),
            knn_q2r,
            knn_r2q)


# speedup vs baseline: 4.1898x; 4.1898x over previous
import jax
import jax.numpy as jnp
from jax.experimental import pallas as pl
from jax.experimental.pallas import tpu as pltpu

N = 196
D = 768
B = 64
TH = 0.8
K2 = 3
MIN_N = 20
SCALE = 5.0
BIG = 2**30
_DN = (((1,), (1,)), ((), ()))


def _adaptive_mask_col(ent_col, ones_col, ii, jj):
    emin = jnp.min(ent_col)
    emax = jnp.max(ent_col)
    abs_th = emin + (1.0 - TH) * (emax - emin)
    mask_col = ent_col < abs_th
    count = jnp.sum(mask_col.astype(jnp.float32))
    e_i = jax.lax.dot_general(ent_col, ones_col, _DN,
                              preferred_element_type=jnp.float32)
    e_j = jax.lax.dot_general(ones_col, ent_col, _DN,
                              preferred_element_type=jnp.float32)
    smaller = ((e_j < e_i) | ((e_j == e_i) & (jj < ii))).astype(jnp.float32)
    ranks_col = jax.lax.dot_general(smaller, ones_col, (((1,), (0,)), ((), ())),
                                    preferred_element_type=jnp.float32)
    fallback = (ranks_col < float(MIN_N)).astype(jnp.int32)
    return jnp.where(count < float(MIN_N), fallback, mask_col.astype(jnp.int32))


def _top3(aff, axis):
    idx_iota = jax.lax.broadcasted_iota(jnp.int32, (N, N), axis)
    work = aff
    cols = []
    for _ in range(K2):
        m = jnp.max(work, axis=axis, keepdims=True)
        idx = jnp.min(jnp.where(work == m, idx_iota, BIG), axis=axis)
        cols.append(idx[:, None])
        sel = idx[:, None] if axis == 1 else idx[None, :]
        work = jnp.where(idx_iota == sel, -jnp.inf, work)
    return jnp.concatenate(cols, axis=1)


def _image_kernel(q_ref, r_ref, aff_ref, entq_ref, entr_ref,
                  maskq_ref, maskr_ref, knnq_ref, knnr_ref):
    q = q_ref[0]
    r = r_ref[0]
    qn = q / jnp.maximum(jnp.sqrt(jnp.sum(q * q, axis=1, keepdims=True)), 1e-12)
    rn = r / jnp.maximum(jnp.sqrt(jnp.sum(r * r, axis=1, keepdims=True)), 1e-12)
    aff = jax.lax.dot_general(qn, rn, _DN,
                              preferred_element_type=jnp.float32)
    aff_ref[0] = aff

    ones_col = jnp.ones((N, 1), jnp.float32)
    ii = jax.lax.broadcasted_iota(jnp.int32, (N, N), 0)
    jj = jax.lax.broadcasted_iota(jnp.int32, (N, N), 1)

    e = jnp.exp(aff * SCALE)
    ae = aff * e
    sq = jnp.sum(e, axis=1, keepdims=True)
    wq = jnp.sum(ae, axis=1, keepdims=True)
    dn0 = (((0,), (0,)), ((), ()))
    sr = jax.lax.dot_general(e, ones_col, dn0,
                             preferred_element_type=jnp.float32)
    wr = jax.lax.dot_general(ae, ones_col, dn0,
                             preferred_element_type=jnp.float32)
    ent_q = jnp.log(sq) - SCALE * wq / sq
    ent_r = jnp.log(sr) - SCALE * wr / sr
    entq_ref[0] = ent_q
    entr_ref[0] = ent_r

    maskq_ref[0] = _adaptive_mask_col(ent_q, ones_col, ii, jj)
    maskr_ref[0] = _adaptive_mask_col(ent_r, ones_col, ii, jj)

    knnq_ref[0] = _top3(aff, axis=1)
    knnr_ref[0] = _top3(aff, axis=0)


@jax.jit
def kernel(que_tokens, ref_tokens):
    grid = (B,)
    in_specs = [
        pl.BlockSpec((1, N, D), lambda b: (b, 0, 0)),
        pl.BlockSpec((1, N, D), lambda b: (b, 0, 0)),
    ]
    out_specs = [
        pl.BlockSpec((1, N, N), lambda b: (b, 0, 0)),
        pl.BlockSpec((1, N, 1), lambda b: (b, 0, 0)),
        pl.BlockSpec((1, N, 1), lambda b: (b, 0, 0)),
        pl.BlockSpec((1, N, 1), lambda b: (b, 0, 0)),
        pl.BlockSpec((1, N, 1), lambda b: (b, 0, 0)),
        pl.BlockSpec((1, N, K2), lambda b: (b, 0, 0)),
        pl.BlockSpec((1, N, K2), lambda b: (b, 0, 0)),
    ]
    out_shapes = [
        jax.ShapeDtypeStruct((B, N, N), jnp.float32),
        jax.ShapeDtypeStruct((B, N, 1), jnp.float32),
        jax.ShapeDtypeStruct((B, N, 1), jnp.float32),
        jax.ShapeDtypeStruct((B, N, 1), jnp.int32),
        jax.ShapeDtypeStruct((B, N, 1), jnp.int32),
        jax.ShapeDtypeStruct((B, N, K2), jnp.int32),
        jax.ShapeDtypeStruct((B, N, K2), jnp.int32),
    ]
    aff, ent_q, ent_r, mask_q, mask_r, knn_q2r, knn_r2q = pl.pallas_call(
        _image_kernel,
        grid=grid,
        in_specs=in_specs,
        out_specs=out_specs,
        out_shape=out_shapes,
        compiler_params=pltpu.CompilerParams(
            dimension_semantics=("parallel",),
        ),
    )(que_tokens, ref_tokens)
    return (aff,
            ent_q.reshape(B, N),
            ent_r.reshape(B, N),
            mask_q.reshape(B, N).astype(jnp.bool_),
            mask_r.reshape(B, N).astype(jnp.bool_),
            knn_q2r,
            knn_r2q)
